# Initial kernel scaffold; baseline (speedup 1.0000x reference)
import jax
import jax.numpy as jnp
from jax.experimental import pallas as pl


def kernel(input_token, token_table, pos_table, gamma, beta):
    raise NotImplementedError("write your pallas kernel here")


# speedup vs baseline: 1.5616x; 1.5616x over previous
import functools

import jax
import jax.numpy as jnp
from jax import lax
from jax.experimental import pallas as pl
from jax.experimental.pallas import tpu as pltpu
from jax.experimental.pallas import tpu_sc as plsc

VOCAB = 100000
SEQ = 200
DIM = 64
BATCH = 1024
EPS = 1e-12

L = 16
NC = 2
NS = 16
NW = NC * NS
B_PER_W = BATCH // NW
NV = DIM // L


def _rsqrt(x):
    i = lax.bitcast_convert_type(x, jnp.int32)
    i = jnp.int32(0x5F3759DF) - lax.shift_right_logical(i, 1)
    y = lax.bitcast_convert_type(i, jnp.float32)
    half = jnp.float32(0.5) * x
    for _ in range(4):
        y = y * (jnp.float32(1.5) - half * y * y)
    return y


def _body(tok_hbm, tab_hbm, pos_hbm, gam_hbm, bet_hbm, out_hbm,
          idx_v, rows_v, pos_v, gam_v, bet_v, sem):
    cid = lax.axis_index("c")
    sid = lax.axis_index("s")
    wid = sid * NC + cid
    b0 = wid * B_PER_W

    pltpu.sync_copy(tok_hbm.at[pl.ds(b0, B_PER_W)], idx_v)
    pltpu.sync_copy(pos_hbm, pos_v)
    pltpu.sync_copy(gam_hbm, gam_v)
    pltpu.sync_copy(bet_hbm, bet_v)

    gam = [gam_v[pl.ds(16 * j, L)] for j in range(NV)]
    bet = [bet_v[pl.ds(16 * j, L)] for j in range(NV)]

    inv_d = jnp.float32(1.0 / DIM)
    iota = lax.iota(jnp.int32, L)
    perms = [iota ^ jnp.int32(step) for step in (1, 2, 4, 8)]

    dnums = lax.GatherDimensionNumbers(
        offset_dims=(), collapsed_slice_dims=(0,), start_index_map=(0,))

    def lane_perm(v, p):
        return lax.gather(v, p[:, None], dimension_numbers=dnums,
                          slice_sizes=(1,),
                          mode=lax.GatherScatterMode.PROMISE_IN_BOUNDS)

    def allsum(v):
        for p in perms:
            v = v + lane_perm(v, p)
        return v

    def per_row(s, _):
        y = [rows_v[s, pl.ds(16 * j, L)] + pos_v[s, pl.ds(16 * j, L)]
             for j in range(NV)]
        sv = (y[0] + y[1]) + (y[2] + y[3])
        qv = (y[0] * y[0] + y[1] * y[1]) + (y[2] * y[2] + y[3] * y[3])
        tot = allsum(sv)
        tot2 = allsum(qv)
        mean = tot * inv_d
        var = tot2 * inv_d - mean * mean
        rstd = _rsqrt(var + jnp.float32(EPS))
        for j in range(NV):
            scale = gam[j] * rstd
            shift = bet[j] - mean * scale
            rows_v[s, pl.ds(16 * j, L)] = y[j] * scale + shift
        return 0

    def per_batch(bi, _):
        pltpu.async_copy(tab_hbm.at[idx_v.at[bi]], rows_v, sem).wait()
        lax.fori_loop(0, SEQ, per_row, 0)
        pltpu.sync_copy(rows_v, out_hbm.at[b0 + bi])
        return 0

    lax.fori_loop(0, B_PER_W, per_batch, 0)


@jax.jit
def _run(tok, tab, pos, gam, bet):
    mesh = plsc.VectorSubcoreMesh(core_axis_name="c", subcore_axis_name="s")
    k = functools.partial(
        pl.kernel,
        out_type=jax.ShapeDtypeStruct((BATCH, SEQ, DIM), jnp.float32),
        mesh=mesh,
        compiler_params=pltpu.CompilerParams(use_tc_tiling_on_sc=False),
        scratch_types=[
            pltpu.VMEM((B_PER_W, SEQ), jnp.int32),
            pltpu.VMEM((SEQ, DIM), jnp.float32),
            pltpu.VMEM((SEQ, DIM), jnp.float32),
            pltpu.VMEM((DIM,), jnp.float32),
            pltpu.VMEM((DIM,), jnp.float32),
            pltpu.SemaphoreType.DMA,
        ],
    )(_body)
    return k(tok, tab, pos, gam, bet)


def kernel(input_token, token_table, pos_table, gamma, beta):
    tok = jnp.asarray(input_token, jnp.int32)
    return _run(tok, token_table, pos_table, gamma, beta)


# speedup vs baseline: 3.0378x; 1.9452x over previous
import functools

import jax
import jax.numpy as jnp
from jax import lax
from jax.experimental import pallas as pl
from jax.experimental.pallas import tpu as pltpu
from jax.experimental.pallas import tpu_sc as plsc

VOCAB = 100000
SEQ = 200
DIM = 64
BATCH = 1024
EPS = 1e-12

L = 16
NC = 2
NS = 16
NW = NC * NS
B_PER_W = BATCH // NW
NV = DIM // L
UNROLL = 4


def _body(tok_hbm, tab_hbm, pos_hbm, gam_hbm, bet_hbm, out_hbm,
          idx_v, in_a, in_b, out_a, out_b, pos_v, gam_v, bet_v,
          sg_a, sg_b, ss_a, ss_b):
    cid = lax.axis_index("c")
    sid = lax.axis_index("s")
    wid = sid * NC + cid
    b0 = wid * B_PER_W

    pltpu.sync_copy(tok_hbm.at[pl.ds(b0, B_PER_W)], idx_v)
    pltpu.sync_copy(pos_hbm, pos_v)
    pltpu.sync_copy(gam_hbm, gam_v)
    pltpu.sync_copy(bet_hbm, bet_v)

    gam = [gam_v[pl.ds(16 * j, L)] for j in range(NV)]
    bet = [bet_v[pl.ds(16 * j, L)] for j in range(NV)]

    inv_d = jnp.float32(1.0 / DIM)
    eps = jnp.float32(EPS)
    iota = lax.iota(jnp.int32, L)
    perms = [iota ^ jnp.int32(step) for step in (1, 2, 4, 8)]
    dnums = lax.GatherDimensionNumbers(
        offset_dims=(), collapsed_slice_dims=(0,), start_index_map=(0,))

    def allsum(v):
        for p in perms:
            v = v + lax.gather(v, p[:, None], dimension_numbers=dnums,
                               slice_sizes=(1,),
                               mode=lax.GatherScatterMode.PROMISE_IN_BOUNDS)
        return v

    def one_row(src, dst, s):
        y = [src[s, pl.ds(16 * j, L)] + pos_v[s, pl.ds(16 * j, L)]
             for j in range(NV)]
        sv = (y[0] + y[1]) + (y[2] + y[3])
        qv = (y[0] * y[0] + y[1] * y[1]) + (y[2] * y[2] + y[3] * y[3])
        mean = allsum(sv) * inv_d
        var = allsum(qv) * inv_d - mean * mean + eps
        i = lax.bitcast_convert_type(var, jnp.int32)
        i = jnp.int32(0x5F3759DF) - lax.shift_right_logical(i, 1)
        r = lax.bitcast_convert_type(i, jnp.float32)
        half = jnp.float32(0.5) * var
        r = r * (jnp.float32(1.5) - half * r * r)
        r = r * (jnp.float32(1.5) - half * r * r)
        for j in range(NV):
            dst[s, pl.ds(16 * j, L)] = (y[j] - mean) * r * gam[j] + bet[j]

    def compute(src, dst):
        def rows(i, _):
            for k in range(UNROLL):
                one_row(src, dst, i * UNROLL + k)
            return 0
        lax.fori_loop(0, SEQ // UNROLL, rows, 0)

    def g_start(buf, sem, bi):
        pltpu.make_async_copy(tab_hbm.at[idx_v.at[bi]], buf, sem).start()

    def g_wait(buf, sem):
        pltpu.make_async_copy(tab_hbm.at[idx_v.at[0]], buf, sem).wait()

    def s_start(buf, sem, b):
        pltpu.make_async_copy(buf, out_hbm.at[b], sem).start()

    def s_wait(buf, sem):
        pltpu.make_async_copy(buf, out_hbm.at[b0], sem).wait()

    last = jnp.int32(B_PER_W - 1)

    def phase(i, b_off, in_buf, out_buf, sg, ss):
        b = 2 * i + b_off
        g_wait(in_buf, sg)

        @pl.when(i > 0)
        def _():
            s_wait(out_buf, ss)

        compute(in_buf, out_buf)
        g_start(in_buf, sg, jnp.minimum(b + 2, last))
        s_start(out_buf, ss, b0 + b)

    def pair(i, _):
        phase(i, 0, in_a, out_a, sg_a, ss_a)
        phase(i, 1, in_b, out_b, sg_b, ss_b)
        return 0

    g_start(in_a, sg_a, jnp.int32(0))
    g_start(in_b, sg_b, jnp.int32(1))
    lax.fori_loop(0, B_PER_W // 2, pair, 0)
    g_wait(in_a, sg_a)
    g_wait(in_b, sg_b)
    s_wait(out_a, ss_a)
    s_wait(out_b, ss_b)


@jax.jit
def _run(tok, tab, pos, gam, bet):
    mesh = plsc.VectorSubcoreMesh(core_axis_name="c", subcore_axis_name="s")
    k = functools.partial(
        pl.kernel,
        out_type=jax.ShapeDtypeStruct((BATCH, SEQ, DIM), jnp.float32),
        mesh=mesh,
        compiler_params=pltpu.CompilerParams(use_tc_tiling_on_sc=False),
        scratch_types=[
            pltpu.VMEM((B_PER_W, SEQ), jnp.int32),
            pltpu.VMEM((SEQ, DIM), jnp.float32),
            pltpu.VMEM((SEQ, DIM), jnp.float32),
            pltpu.VMEM((SEQ, DIM), jnp.float32),
            pltpu.VMEM((SEQ, DIM), jnp.float32),
            pltpu.VMEM((SEQ, DIM), jnp.float32),
            pltpu.VMEM((DIM,), jnp.float32),
            pltpu.VMEM((DIM,), jnp.float32),
            pltpu.SemaphoreType.DMA,
            pltpu.SemaphoreType.DMA,
            pltpu.SemaphoreType.DMA,
            pltpu.SemaphoreType.DMA,
        ],
    )(_body)
    return k(tok, tab, pos, gam, bet)


def kernel(input_token, token_table, pos_table, gamma, beta):
    tok = jnp.asarray(input_token, jnp.int32)
    return _run(tok, token_table, pos_table, gamma, beta)
